# Initial kernel scaffold; baseline (speedup 1.0000x reference)
import functools

import jax
import jax.numpy as jnp
from jax import lax
from jax.experimental import pallas as pl
from jax.experimental.pallas import tpu as pltpu
from jax.experimental.pallas import tpu_sc as plsc

_SQRT_HALF = 0.7071067811865476

_NC = 2
_NS = 16
_C = 128


def _gelu_exact(x):
    return x * 0.5 * (1.0 + lax.erf(x * _SQRT_HALF))


def _sc_edge_aggregate(h, src, dst):
    n, d = h.shape
    e = src.shape[0]
    nw = _NC * _NS
    nchunks = e // _C
    rows_per_tile = n // _NS

    mesh = plsc.VectorSubcoreMesh(core_axis_name="c", subcore_axis_name="s")

    @functools.partial(
        pl.kernel,
        out_type=(
            jax.ShapeDtypeStruct((_NC, n, d), jnp.float32),
            jax.ShapeDtypeStruct((_NC, n, 16), jnp.float32),
        ),
        mesh=mesh,
        scratch_types=[
            pltpu.VMEM_SHARED((n, d), jnp.float32),
            pltpu.VMEM_SHARED((n, 16), jnp.float32),
            pltpu.VMEM((_C,), jnp.int32),
            pltpu.VMEM((_C,), jnp.int32),
            pltpu.VMEM((_C, d), jnp.float32),
            pltpu.VMEM((_C, 16), jnp.float32),
            pltpu.SemaphoreType.DMA,
        ],
    )
    def k(h_hbm, src_hbm, dst_hbm, agg_out, cnt_out,
          agg_sp, cnt_sp, src_v, dst_v, rows_v, ones_v, sem):
        cc = lax.axis_index("c")
        ss = lax.axis_index("s")
        tid = ss * _NC + cc
        row0 = ss * rows_per_tile

        def zrow(i, carry):
            for j in range(d // 16):
                rows_v[i, pl.ds(j * 16, 16)] = jnp.zeros((16,), jnp.float32)
            ones_v[i] = jnp.zeros((16,), jnp.float32)
            return carry
        lax.fori_loop(0, _C, zrow, 0)
        off = 0
        while off < rows_per_tile:
            sz = min(_C, rows_per_tile - off)
            pltpu.sync_copy(rows_v.at[pl.ds(0, sz)],
                            agg_sp.at[pl.ds(row0 + off, sz)])
            pltpu.sync_copy(ones_v.at[pl.ds(0, sz)],
                            cnt_sp.at[pl.ds(row0 + off, sz)])
            off += sz

        def orow(i, carry):
            ones_v[i] = jnp.ones((16,), jnp.float32)
            return carry
        lax.fori_loop(0, _C, orow, 0)
        plsc.subcore_barrier()

        nfull = nchunks // nw
        extra = nchunks % nw
        niter = jnp.where(tid < extra, nfull + 1, nfull)

        def body(i, carry):
            base = (i * nw + tid) * _C
            pltpu.sync_copy(src_hbm.at[pl.ds(base, _C)], src_v)
            pltpu.sync_copy(dst_hbm.at[pl.ds(base, _C)], dst_v)
            pltpu.async_copy(h_hbm.at[src_v], rows_v, sem).wait()
            pltpu.sync_copy(rows_v, agg_sp.at[dst_v], add=True)
            pltpu.sync_copy(ones_v, cnt_sp.at[dst_v], add=True)
            return carry
        lax.fori_loop(0, niter, body, 0)
        plsc.subcore_barrier()

        pltpu.sync_copy(agg_sp.at[pl.ds(row0, rows_per_tile)],
                        agg_out.at[cc, pl.ds(row0, rows_per_tile)])
        pltpu.sync_copy(cnt_sp.at[pl.ds(row0, rows_per_tile)],
                        cnt_out.at[cc, pl.ds(row0, rows_per_tile)])

    return k(h, src, dst)


def _ln_forward(x, g, b):
    n, d = x.shape
    blk = 1000

    def body(x_ref, g_ref, b_ref, o_ref):
        xv = x_ref[...]
        m = jnp.mean(xv, axis=1, keepdims=True)
        v = jnp.mean((xv - m) ** 2, axis=1, keepdims=True)
        o_ref[...] = (xv - m) * lax.rsqrt(v + 1e-5) * g_ref[...] + b_ref[...]

    return pl.pallas_call(
        body,
        grid=(n // blk,),
        in_specs=[
            pl.BlockSpec((blk, d), lambda i: (i, 0)),
            pl.BlockSpec((1, d), lambda i: (0, 0)),
            pl.BlockSpec((1, d), lambda i: (0, 0)),
        ],
        out_specs=pl.BlockSpec((blk, d), lambda i: (i, 0)),
        out_shape=jax.ShapeDtypeStruct((n, d), jnp.float32),
    )(x, g, b)


def _conv_gate(h, aggp, cntp, batch2, llw, llb, lrw, lng, lnb, gw, gb, ng):
    n, d = h.shape
    blk = 1000
    nblk = n // blk

    def body(batch_ref, h_ref, a0_ref, a1_ref, c0_ref, c1_ref,
             llw_ref, llb_ref, lrw_ref, lng_ref, lnb_ref, gw_ref, gb_ref,
             h3_ref, gate_ref, gmax_ref):
        i = pl.program_id(0)
        cnt = c0_ref[0][:, :1] + c1_ref[0][:, :1]
        mean = (a0_ref[0] + a1_ref[0]) / jnp.maximum(cnt, 1.0)
        h2 = (jnp.dot(mean, llw_ref[...], preferred_element_type=jnp.float32)
              + jnp.dot(h_ref[...], lrw_ref[...], preferred_element_type=jnp.float32)
              + llb_ref[...])
        m = jnp.mean(h2, axis=1, keepdims=True)
        v = jnp.mean((h2 - m) ** 2, axis=1, keepdims=True)
        h2n = (h2 - m) * lax.rsqrt(v + 1e-5) * lng_ref[...] + lnb_ref[...]
        h3 = _gelu_exact(h2n)
        h3_ref[...] = h3
        gate = jnp.dot(h3, gw_ref[...], preferred_element_type=jnp.float32) + gb_ref[...]
        gate_ref[...] = gate

        @pl.when(i == 0)
        def _():
            gmax_ref[...] = jnp.full(gmax_ref.shape, -jnp.inf, jnp.float32)
        onehot = batch_ref[...] == lax.broadcasted_iota(jnp.int32, (1, ng), 1)
        masked = jnp.where(onehot, gate, -jnp.inf)
        gmax_ref[...] = jnp.maximum(gmax_ref[...],
                                    jnp.max(masked, axis=0, keepdims=True))

    return pl.pallas_call(
        body,
        grid=(nblk,),
        in_specs=[
            pl.BlockSpec((blk, 1), lambda i: (i, 0)),
            pl.BlockSpec((blk, d), lambda i: (i, 0)),
            pl.BlockSpec((1, blk, d), lambda i: (0, i, 0)),
            pl.BlockSpec((1, blk, d), lambda i: (1, i, 0)),
            pl.BlockSpec((1, blk, 16), lambda i: (0, i, 0)),
            pl.BlockSpec((1, blk, 16), lambda i: (1, i, 0)),
            pl.BlockSpec((d, d), lambda i: (0, 0)),
            pl.BlockSpec((1, d), lambda i: (0, 0)),
            pl.BlockSpec((d, d), lambda i: (0, 0)),
            pl.BlockSpec((1, d), lambda i: (0, 0)),
            pl.BlockSpec((1, d), lambda i: (0, 0)),
            pl.BlockSpec((d, 1), lambda i: (0, 0)),
            pl.BlockSpec((1, 1), lambda i: (0, 0)),
        ],
        out_specs=[
            pl.BlockSpec((blk, d), lambda i: (i, 0)),
            pl.BlockSpec((blk, 1), lambda i: (i, 0)),
            pl.BlockSpec((1, ng), lambda i: (0, 0)),
        ],
        out_shape=[
            jax.ShapeDtypeStruct((n, d), jnp.float32),
            jax.ShapeDtypeStruct((n, 1), jnp.float32),
            jax.ShapeDtypeStruct((1, ng), jnp.float32),
        ],
    )(batch2, h, aggp, aggp, cntp, cntp, llw, llb, lrw, lng, lnb, gw, gb)


def _softmax_accum(batch2, gate, h3, gmax, ng):
    n, d = h3.shape
    blk = 1000
    nblk = n // blk

    def body(batch_ref, gate_ref, h3_ref, gmax_ref, e_ref, sext_ref):
        i = pl.program_id(0)

        @pl.when(i == 0)
        def _():
            sext_ref[...] = jnp.zeros(sext_ref.shape, jnp.float32)
        onehot = batch_ref[...] == lax.broadcasted_iota(jnp.int32, (1, ng), 1)
        gsel = jnp.max(jnp.where(onehot, gmax_ref[...], -jnp.inf),
                       axis=1, keepdims=True)
        ev = jnp.exp(gate_ref[...] - gsel)
        e_ref[...] = ev
        ohf = onehot.astype(jnp.float32)
        ehe = jnp.concatenate([ev * h3_ref[...], ev], axis=1)
        sext_ref[...] += lax.dot_general(
            ohf, ehe, (((0,), (0,)), ((), ())),
            preferred_element_type=jnp.float32)

    return pl.pallas_call(
        body,
        grid=(nblk,),
        in_specs=[
            pl.BlockSpec((blk, 1), lambda i: (i, 0)),
            pl.BlockSpec((blk, 1), lambda i: (i, 0)),
            pl.BlockSpec((blk, d), lambda i: (i, 0)),
            pl.BlockSpec((1, ng), lambda i: (0, 0)),
        ],
        out_specs=[
            pl.BlockSpec((blk, 1), lambda i: (i, 0)),
            pl.BlockSpec((ng, d + 1), lambda i: (0, 0)),
        ],
        out_shape=[
            jax.ShapeDtypeStruct((n, 1), jnp.float32),
            jax.ShapeDtypeStruct((ng, d + 1), jnp.float32),
        ],
    )(batch2, gate, h3, gmax)


def _head_attn(e, batch2, sext, text, feats, mw, mb, fw, fb, mg, mbb,
               f1w, f1b, gwt, ng):
    n = e.shape[0]
    d = sext.shape[1] - 1
    dt = text.shape[1]
    df = feats.shape[1]
    blk = 1000
    nblk = n // blk

    def body(batch_ref, e_ref, sext_ref, text_ref, feats_ref,
             mw_ref, mb_ref, fw_ref, fb_ref, mg_ref, mbb_ref,
             f1w_ref, f1b_ref, gwt_ref,
             logits_ref, gemb_ref, attn_ref):
        i = pl.program_id(0)
        den = sext_ref[...][:, d:d + 1]
        rec = 1.0 / (den + 1e-16)

        @pl.when(i == 0)
        def _():
            ge = sext_ref[...][:, :d] * rec
            gemb_ref[...] = ge
            wg = gwt_ref[0, 0] * ge
            msg = _gelu_exact(
                jnp.dot(text_ref[...], mw_ref[...],
                        preferred_element_type=jnp.float32) + mb_ref[...])
            ft = _gelu_exact(
                jnp.dot(feats_ref[...], fw_ref[...],
                        preferred_element_type=jnp.float32) + fb_ref[...])
            emb = jnp.concatenate([wg, msg, ft], axis=1)
            m = jnp.mean(emb, axis=1, keepdims=True)
            v = jnp.mean((emb - m) ** 2, axis=1, keepdims=True)
            embn = (emb - m) * lax.rsqrt(v + 1e-5) * mg_ref[...] + mbb_ref[...]
            logits_ref[...] = jnp.dot(
                embn, f1w_ref[...], preferred_element_type=jnp.float32) + f1b_ref[...]

        ohf = (batch_ref[...] ==
               lax.broadcasted_iota(jnp.int32, (1, ng), 1)).astype(jnp.float32)
        attn_ref[...] = e_ref[...] * jnp.dot(
            ohf, rec, preferred_element_type=jnp.float32)

    return pl.pallas_call(
        body,
        grid=(nblk,),
        in_specs=[
            pl.BlockSpec((blk, 1), lambda i: (i, 0)),
            pl.BlockSpec((blk, 1), lambda i: (i, 0)),
            pl.BlockSpec((ng, d + 1), lambda i: (0, 0)),
            pl.BlockSpec((ng, dt), lambda i: (0, 0)),
            pl.BlockSpec((ng, df), lambda i: (0, 0)),
            pl.BlockSpec((dt, d), lambda i: (0, 0)),
            pl.BlockSpec((1, d), lambda i: (0, 0)),
            pl.BlockSpec((df, d), lambda i: (0, 0)),
            pl.BlockSpec((1, d), lambda i: (0, 0)),
            pl.BlockSpec((1, 3 * d), lambda i: (0, 0)),
            pl.BlockSpec((1, 3 * d), lambda i: (0, 0)),
            pl.BlockSpec((3 * d, 1), lambda i: (0, 0)),
            pl.BlockSpec((1, 1), lambda i: (0, 0)),
            pl.BlockSpec((1, 1), lambda i: (0, 0)),
        ],
        out_specs=[
            pl.BlockSpec((ng, 1), lambda i: (0, 0)),
            pl.BlockSpec((ng, d), lambda i: (0, 0)),
            pl.BlockSpec((blk, 1), lambda i: (i, 0)),
        ],
        out_shape=[
            jax.ShapeDtypeStruct((ng, 1), jnp.float32),
            jax.ShapeDtypeStruct((ng, d), jnp.float32),
            jax.ShapeDtypeStruct((n, 1), jnp.float32),
        ],
    )(batch2, e, sext, text, feats, mw, mb, fw, fb, mg, mbb, f1w, f1b, gwt)


def kernel(x_dict, edge_index, batch, batch_size, text_embedding,
           features_embedding, pn_g, pn_b, lin_l_w, lin_l_b, lin_r_w,
           ln_g, ln_b, gate_w, gate_b, graph_weight, msg_w, msg_b,
           feat_w, feat_b, mix_g, mix_b, fc1_w, fc1_b):
    n, d = x_dict.shape
    ng = text_embedding.shape[0]

    h = _ln_forward(x_dict, pn_g.reshape(1, d), pn_b.reshape(1, d))

    src = edge_index[0]
    dst = edge_index[1]
    aggp, cntp = _sc_edge_aggregate(h, src, dst)

    batch2 = batch.reshape(n, 1)
    h3, gate, gmax = _conv_gate(
        h, aggp, cntp, batch2, lin_l_w, lin_l_b.reshape(1, d), lin_r_w,
        ln_g.reshape(1, d), ln_b.reshape(1, d), gate_w,
        gate_b.reshape(1, 1), ng)

    e, sext = _softmax_accum(batch2, gate, h3, gmax, ng)

    logits, graph_emb, attn = _head_attn(
        e, batch2, sext, text_embedding, features_embedding,
        msg_w, msg_b.reshape(1, d), feat_w, feat_b.reshape(1, d),
        mix_g.reshape(1, 3 * d), mix_b.reshape(1, 3 * d), fc1_w,
        fc1_b.reshape(1, 1), jnp.reshape(graph_weight, (1, 1)), ng)

    return logits, graph_emb, attn


# speedup vs baseline: 5.9513x; 5.9513x over previous
import functools

import jax
import jax.numpy as jnp
from jax import lax
from jax.experimental import pallas as pl
from jax.experimental.pallas import tpu as pltpu
from jax.experimental.pallas import tpu_sc as plsc

_SQRT_HALF = 0.7071067811865476

_NC = 2
_NS = 16
_C = 80


def _gelu_exact(x):
    return x * 0.5 * (1.0 + lax.erf(x * _SQRT_HALF))


def _sc_edge_aggregate(h, src, dst):
    n, d = h.shape
    e = src.shape[0]
    nw = _NC * _NS
    per_tile = e // nw
    niter = per_tile // _C
    rows_per_tile = (n // _NS) // 8 * 8
    rows_rem = n - _NS * rows_per_tile
    pack_rows = 8
    npad = _NS * pack_rows
    nvals_max = rows_per_tile + rows_rem

    mesh = plsc.VectorSubcoreMesh(core_axis_name="c", subcore_axis_name="s")

    @functools.partial(
        pl.kernel,
        out_type=jax.ShapeDtypeStruct((_NC * (n + npad), d), jnp.float32),
        mesh=mesh,
        scratch_types=[
            pltpu.VMEM_SHARED((n + npad, d), jnp.float32),
            pltpu.VMEM_SHARED((n,), jnp.float32),
            pltpu.VMEM((_C,), jnp.int32),
            pltpu.VMEM((_C,), jnp.int32),
            pltpu.VMEM((_C, d), jnp.float32),
            pltpu.VMEM((_C,), jnp.float32),
            pltpu.VMEM((nvals_max,), jnp.float32),
            pltpu.VMEM((pack_rows, d), jnp.float32),
            pltpu.SemaphoreType.DMA,
        ],
    )
    def k(h_hbm, src_hbm, dst_hbm, agg_out,
          agg_sp, cnt_sp, src_v, dst_v, rows_v, ones_v, cbuf, pbuf, sem):
        cc = lax.axis_index("c")
        ss = lax.axis_index("s")
        tid = ss * _NC + cc
        tile_base = tid * per_tile
        row0 = ss * rows_per_tile

        def zrow(i, carry):
            for j in range(d // 16):
                rows_v[i, pl.ds(j * 16, 16)] = jnp.zeros((16,), jnp.float32)
            return carry
        lax.fori_loop(0, _C, zrow, 0)
        def zc(i, carry):
            cbuf[pl.ds(i * 16, 16)] = jnp.zeros((16,), jnp.float32)
            return carry
        lax.fori_loop(0, nvals_max // 16, zc, 0)
        def orow(i, carry):
            ones_v[pl.ds(i * 16, 16)] = jnp.ones((16,), jnp.float32)
            return carry
        lax.fori_loop(0, _C // 16, orow, 0)
        off = 0
        while off < rows_per_tile:
            sz = min(_C, rows_per_tile - off)
            pltpu.sync_copy(rows_v.at[pl.ds(0, sz)],
                            agg_sp.at[pl.ds(row0 + off, sz)])
            off += sz
        pltpu.sync_copy(cbuf.at[pl.ds(0, rows_per_tile)],
                        cnt_sp.at[pl.ds(row0, rows_per_tile)])
        if rows_rem:
            @pl.when(ss == _NS - 1)
            def _():
                pltpu.sync_copy(rows_v.at[pl.ds(0, rows_rem)],
                                agg_sp.at[pl.ds(_NS * rows_per_tile, rows_rem)])
                pltpu.sync_copy(cbuf.at[pl.ds(0, rows_rem)],
                                cnt_sp.at[pl.ds(_NS * rows_per_tile, rows_rem)])
        plsc.subcore_barrier()

        def body(i, carry):
            base = tile_base + i * _C
            pltpu.sync_copy(src_hbm.at[pl.ds(base, _C)], src_v)
            pltpu.sync_copy(dst_hbm.at[pl.ds(base, _C)], dst_v)
            pltpu.async_copy(h_hbm.at[src_v], rows_v, sem).wait()
            pltpu.sync_copy(rows_v, agg_sp.at[dst_v], add=True)
            pltpu.sync_copy(ones_v, cnt_sp.at[dst_v], add=True)
            return carry
        lax.fori_loop(0, niter, body, 0)
        plsc.subcore_barrier()

        pltpu.sync_copy(cnt_sp.at[pl.ds(row0, rows_per_tile)],
                        cbuf.at[pl.ds(0, rows_per_tile)])
        if rows_rem:
            @pl.when(ss == _NS - 1)
            def _():
                pltpu.sync_copy(cnt_sp.at[pl.ds(_NS * rows_per_tile, rows_rem)],
                                cbuf.at[pl.ds(rows_per_tile, rows_rem)])
        for kk in range(nvals_max // 16):
            pbuf[kk >> 3, pl.ds((kk & 7) * 16, 16)] = cbuf[pl.ds(kk * 16, 16)]
        pltpu.sync_copy(pbuf, agg_sp.at[pl.ds(n + ss * pack_rows, pack_rows)])

        out0 = cc * (n + npad)
        pltpu.sync_copy(agg_sp.at[pl.ds(row0, rows_per_tile)],
                        agg_out.at[pl.ds(out0 + row0, rows_per_tile)])
        if rows_rem:
            @pl.when(ss == _NS - 1)
            def _():
                base = _NS * rows_per_tile
                pltpu.sync_copy(agg_sp.at[pl.ds(base, rows_rem)],
                                agg_out.at[pl.ds(out0 + base, rows_rem)])
        pltpu.sync_copy(agg_sp.at[pl.ds(n + ss * pack_rows, pack_rows)],
                        agg_out.at[pl.ds(out0 + n + ss * pack_rows, pack_rows)])

    aggf = k(h, src, dst).reshape(_NC, n + npad, d)
    aggp = aggf[:, :n]
    packed = aggf[:, n:].reshape(_NC, _NS, pack_rows * d)
    cnt = jnp.concatenate(
        [packed[:, :_NS - 1, :rows_per_tile].reshape(_NC, -1),
         packed[:, _NS - 1, :rows_per_tile + rows_rem]], axis=1)
    return aggp, cnt[..., None]


def _ln_forward(x, g, b):
    n, d = x.shape
    blk = 1000

    def body(x_ref, g_ref, b_ref, o_ref):
        xv = x_ref[...]
        m = jnp.mean(xv, axis=1, keepdims=True)
        v = jnp.mean((xv - m) ** 2, axis=1, keepdims=True)
        o_ref[...] = (xv - m) * lax.rsqrt(v + 1e-5) * g_ref[...] + b_ref[...]

    return pl.pallas_call(
        body,
        grid=(n // blk,),
        in_specs=[
            pl.BlockSpec((blk, d), lambda i: (i, 0)),
            pl.BlockSpec((1, d), lambda i: (0, 0)),
            pl.BlockSpec((1, d), lambda i: (0, 0)),
        ],
        out_specs=pl.BlockSpec((blk, d), lambda i: (i, 0)),
        out_shape=jax.ShapeDtypeStruct((n, d), jnp.float32),
    )(x, g, b)


def _conv_gate(h, aggp, cntp, batch2, llw, llb, lrw, lng, lnb, gw, gb, ng):
    n, d = h.shape
    blk = 1000
    nblk = n // blk

    def body(batch_ref, h_ref, a0_ref, a1_ref, c0_ref, c1_ref,
             llw_ref, llb_ref, lrw_ref, lng_ref, lnb_ref, gw_ref, gb_ref,
             h3_ref, gate_ref, gmax_ref):
        i = pl.program_id(0)
        cnt = c0_ref[0] + c1_ref[0]
        mean = (a0_ref[0] + a1_ref[0]) / jnp.maximum(cnt, 1.0)
        h2 = (jnp.dot(mean, llw_ref[...], preferred_element_type=jnp.float32)
              + jnp.dot(h_ref[...], lrw_ref[...], preferred_element_type=jnp.float32)
              + llb_ref[...])
        m = jnp.mean(h2, axis=1, keepdims=True)
        v = jnp.mean((h2 - m) ** 2, axis=1, keepdims=True)
        h2n = (h2 - m) * lax.rsqrt(v + 1e-5) * lng_ref[...] + lnb_ref[...]
        h3 = _gelu_exact(h2n)
        h3_ref[...] = h3
        gate = jnp.dot(h3, gw_ref[...], preferred_element_type=jnp.float32) + gb_ref[...]
        gate_ref[...] = gate

        @pl.when(i == 0)
        def _():
            gmax_ref[...] = jnp.full(gmax_ref.shape, -jnp.inf, jnp.float32)
        onehot = batch_ref[...] == lax.broadcasted_iota(jnp.int32, (1, ng), 1)
        masked = jnp.where(onehot, gate, -jnp.inf)
        gmax_ref[...] = jnp.maximum(gmax_ref[...],
                                    jnp.max(masked, axis=0, keepdims=True))

    return pl.pallas_call(
        body,
        grid=(nblk,),
        in_specs=[
            pl.BlockSpec((blk, 1), lambda i: (i, 0)),
            pl.BlockSpec((blk, d), lambda i: (i, 0)),
            pl.BlockSpec((1, blk, d), lambda i: (0, i, 0)),
            pl.BlockSpec((1, blk, d), lambda i: (1, i, 0)),
            pl.BlockSpec((1, blk, 1), lambda i: (0, i, 0)),
            pl.BlockSpec((1, blk, 1), lambda i: (1, i, 0)),
            pl.BlockSpec((d, d), lambda i: (0, 0)),
            pl.BlockSpec((1, d), lambda i: (0, 0)),
            pl.BlockSpec((d, d), lambda i: (0, 0)),
            pl.BlockSpec((1, d), lambda i: (0, 0)),
            pl.BlockSpec((1, d), lambda i: (0, 0)),
            pl.BlockSpec((d, 1), lambda i: (0, 0)),
            pl.BlockSpec((1, 1), lambda i: (0, 0)),
        ],
        out_specs=[
            pl.BlockSpec((blk, d), lambda i: (i, 0)),
            pl.BlockSpec((blk, 1), lambda i: (i, 0)),
            pl.BlockSpec((1, ng), lambda i: (0, 0)),
        ],
        out_shape=[
            jax.ShapeDtypeStruct((n, d), jnp.float32),
            jax.ShapeDtypeStruct((n, 1), jnp.float32),
            jax.ShapeDtypeStruct((1, ng), jnp.float32),
        ],
    )(batch2, h, aggp, aggp, cntp, cntp, llw, llb, lrw, lng, lnb, gw, gb)


def _softmax_accum(batch2, gate, h3, gmax, ng):
    n, d = h3.shape
    blk = 1000
    nblk = n // blk

    def body(batch_ref, gate_ref, h3_ref, gmax_ref, e_ref, sext_ref):
        i = pl.program_id(0)

        @pl.when(i == 0)
        def _():
            sext_ref[...] = jnp.zeros(sext_ref.shape, jnp.float32)
        onehot = batch_ref[...] == lax.broadcasted_iota(jnp.int32, (1, ng), 1)
        gsel = jnp.max(jnp.where(onehot, gmax_ref[...], -jnp.inf),
                       axis=1, keepdims=True)
        ev = jnp.exp(gate_ref[...] - gsel)
        e_ref[...] = ev
        ohf = onehot.astype(jnp.float32)
        ehe = jnp.concatenate([ev * h3_ref[...], ev], axis=1)
        sext_ref[...] += lax.dot_general(
            ohf, ehe, (((0,), (0,)), ((), ())),
            preferred_element_type=jnp.float32)

    return pl.pallas_call(
        body,
        grid=(nblk,),
        in_specs=[
            pl.BlockSpec((blk, 1), lambda i: (i, 0)),
            pl.BlockSpec((blk, 1), lambda i: (i, 0)),
            pl.BlockSpec((blk, d), lambda i: (i, 0)),
            pl.BlockSpec((1, ng), lambda i: (0, 0)),
        ],
        out_specs=[
            pl.BlockSpec((blk, 1), lambda i: (i, 0)),
            pl.BlockSpec((ng, d + 1), lambda i: (0, 0)),
        ],
        out_shape=[
            jax.ShapeDtypeStruct((n, 1), jnp.float32),
            jax.ShapeDtypeStruct((ng, d + 1), jnp.float32),
        ],
    )(batch2, gate, h3, gmax)


def _head_attn(e, batch2, sext, text, feats, mw, mb, fw, fb, mg, mbb,
               f1w, f1b, gwt, ng):
    n = e.shape[0]
    d = sext.shape[1] - 1
    dt = text.shape[1]
    df = feats.shape[1]
    blk = 1000
    nblk = n // blk

    def body(batch_ref, e_ref, sext_ref, text_ref, feats_ref,
             mw_ref, mb_ref, fw_ref, fb_ref, mg_ref, mbb_ref,
             f1w_ref, f1b_ref, gwt_ref,
             logits_ref, gemb_ref, attn_ref):
        i = pl.program_id(0)
        den = sext_ref[...][:, d:d + 1]
        rec = 1.0 / (den + 1e-16)

        @pl.when(i == 0)
        def _():
            ge = sext_ref[...][:, :d] * rec
            gemb_ref[...] = ge
            wg = gwt_ref[0, 0] * ge
            msg = _gelu_exact(
                jnp.dot(text_ref[...], mw_ref[...],
                        preferred_element_type=jnp.float32) + mb_ref[...])
            ft = _gelu_exact(
                jnp.dot(feats_ref[...], fw_ref[...],
                        preferred_element_type=jnp.float32) + fb_ref[...])
            emb = jnp.concatenate([wg, msg, ft], axis=1)
            m = jnp.mean(emb, axis=1, keepdims=True)
            v = jnp.mean((emb - m) ** 2, axis=1, keepdims=True)
            embn = (emb - m) * lax.rsqrt(v + 1e-5) * mg_ref[...] + mbb_ref[...]
            logits_ref[...] = jnp.dot(
                embn, f1w_ref[...], preferred_element_type=jnp.float32) + f1b_ref[...]

        ohf = (batch_ref[...] ==
               lax.broadcasted_iota(jnp.int32, (1, ng), 1)).astype(jnp.float32)
        attn_ref[...] = e_ref[...] * jnp.dot(
            ohf, rec, preferred_element_type=jnp.float32)

    return pl.pallas_call(
        body,
        grid=(nblk,),
        in_specs=[
            pl.BlockSpec((blk, 1), lambda i: (i, 0)),
            pl.BlockSpec((blk, 1), lambda i: (i, 0)),
            pl.BlockSpec((ng, d + 1), lambda i: (0, 0)),
            pl.BlockSpec((ng, dt), lambda i: (0, 0)),
            pl.BlockSpec((ng, df), lambda i: (0, 0)),
            pl.BlockSpec((dt, d), lambda i: (0, 0)),
            pl.BlockSpec((1, d), lambda i: (0, 0)),
            pl.BlockSpec((df, d), lambda i: (0, 0)),
            pl.BlockSpec((1, d), lambda i: (0, 0)),
            pl.BlockSpec((1, 3 * d), lambda i: (0, 0)),
            pl.BlockSpec((1, 3 * d), lambda i: (0, 0)),
            pl.BlockSpec((3 * d, 1), lambda i: (0, 0)),
            pl.BlockSpec((1, 1), lambda i: (0, 0)),
            pl.BlockSpec((1, 1), lambda i: (0, 0)),
        ],
        out_specs=[
            pl.BlockSpec((ng, 1), lambda i: (0, 0)),
            pl.BlockSpec((ng, d), lambda i: (0, 0)),
            pl.BlockSpec((blk, 1), lambda i: (i, 0)),
        ],
        out_shape=[
            jax.ShapeDtypeStruct((ng, 1), jnp.float32),
            jax.ShapeDtypeStruct((ng, d), jnp.float32),
            jax.ShapeDtypeStruct((n, 1), jnp.float32),
        ],
    )(batch2, e, sext, text, feats, mw, mb, fw, fb, mg, mbb, f1w, f1b, gwt)


def kernel(x_dict, edge_index, batch, batch_size, text_embedding,
           features_embedding, pn_g, pn_b, lin_l_w, lin_l_b, lin_r_w,
           ln_g, ln_b, gate_w, gate_b, graph_weight, msg_w, msg_b,
           feat_w, feat_b, mix_g, mix_b, fc1_w, fc1_b):
    n, d = x_dict.shape
    ng = text_embedding.shape[0]

    h = _ln_forward(x_dict, pn_g.reshape(1, d), pn_b.reshape(1, d))

    src = edge_index[0]
    dst = edge_index[1]
    aggp, cntp = _sc_edge_aggregate(h, src, dst)

    batch2 = batch.reshape(n, 1)
    h3, gate, gmax = _conv_gate(
        h, aggp, cntp, batch2, lin_l_w, lin_l_b.reshape(1, d), lin_r_w,
        ln_g.reshape(1, d), ln_b.reshape(1, d), gate_w,
        gate_b.reshape(1, 1), ng)

    e, sext = _softmax_accum(batch2, gate, h3, gmax, ng)

    logits, graph_emb, attn = _head_attn(
        e, batch2, sext, text_embedding, features_embedding,
        msg_w, msg_b.reshape(1, d), feat_w, feat_b.reshape(1, d),
        mix_g.reshape(1, 3 * d), mix_b.reshape(1, 3 * d), fc1_w,
        fc1_b.reshape(1, 1), jnp.reshape(graph_weight, (1, 1)), ng)

    return logits, graph_emb, attn
